# Initial kernel scaffold; baseline (speedup 1.0000x reference)
import jax
import jax.numpy as jnp
from jax.experimental import pallas as pl


def kernel(x, Wr, br, W1, b1, W2, b2):
    raise NotImplementedError("write your pallas kernel here")


# speedup vs baseline: 1.5092x; 1.5092x over previous
import jax
import jax.numpy as jnp
from jax.experimental import pallas as pl

D_MODEL = 768
NUM_EXPERTS = 8
EXPERT_HIDDEN = 2048
S = 2048


def _moe_dense_kernel(x_ref, wr_ref, br_ref, w1_ref, b1_ref, w2_ref, b2_ref,
                      out_ref):
    e = pl.program_id(0)
    x = x_ref[...]
    logits = jnp.dot(x, wr_ref[...],
                     preferred_element_type=jnp.float32) + br_ref[...]
    l1 = jnp.max(logits, axis=-1, keepdims=True)
    e1 = jnp.argmax(logits, axis=-1)[:, None]
    cols = jax.lax.broadcasted_iota(jnp.int32, logits.shape, 1)
    masked = jnp.where(cols == e1, -jnp.inf, logits)
    l2 = jnp.max(masked, axis=-1, keepdims=True)
    e2 = jnp.argmax(masked, axis=-1)[:, None]
    s1 = 1.0 / (1.0 + jnp.exp(l2 - l1))
    s2 = 1.0 - s1
    w = jnp.where(e1 == e, s1, jnp.where(e2 == e, s2, 0.0))

    h = jnp.maximum(
        jnp.dot(x, w1_ref[0], preferred_element_type=jnp.float32) + b1_ref[0, 0],
        0.0)
    y = jnp.dot(h, w2_ref[0], preferred_element_type=jnp.float32) + b2_ref[0, 0]
    contrib = w * y

    @pl.when(e == 0)
    def _():
        out_ref[...] = contrib

    @pl.when(e > 0)
    def _():
        out_ref[...] = out_ref[...] + contrib


def kernel(x, Wr, br, W1, b1, W2, b2):
    xs = x.reshape(S, D_MODEL)
    out = pl.pallas_call(
        _moe_dense_kernel,
        grid=(NUM_EXPERTS,),
        in_specs=[
            pl.BlockSpec((S, D_MODEL), lambda e: (0, 0)),
            pl.BlockSpec((D_MODEL, NUM_EXPERTS), lambda e: (0, 0)),
            pl.BlockSpec((NUM_EXPERTS,), lambda e: (0,)),
            pl.BlockSpec((1, D_MODEL, EXPERT_HIDDEN), lambda e: (e, 0, 0)),
            pl.BlockSpec((1, 1, EXPERT_HIDDEN), lambda e: (e, 0, 0)),
            pl.BlockSpec((1, EXPERT_HIDDEN, D_MODEL), lambda e: (e, 0, 0)),
            pl.BlockSpec((1, 1, D_MODEL), lambda e: (e, 0, 0)),
        ],
        out_specs=pl.BlockSpec((S, D_MODEL), lambda e: (0, 0)),
        out_shape=jax.ShapeDtypeStruct((S, D_MODEL), jnp.float32),
    )(xs, Wr, br, W1, b1.reshape(NUM_EXPERTS, 1, EXPERT_HIDDEN),
      W2, b2.reshape(NUM_EXPERTS, 1, D_MODEL))
    return out.reshape(x.shape)
